# baseline (speedup 1.0000x reference)
import jax
import jax.numpy as jnp
import numpy as np
from jax import lax
from jax.experimental import pallas as pl
from jax.experimental.pallas import tpu as pltpu
from jax.experimental.pallas import tpu_sc as plsc

_K = 64
_IMIN = np.int32(-2147483648)


def _keys(v):
    bits = lax.bitcast_convert_type(v, jnp.int32)
    s = bits ^ (lax.shift_right_arithmetic(bits, 31) & np.int32(0x7FFFFFFF))
    return s


def _sc_body(x_hbm, o_hbm, xv, ov, hist, sem):
    nc = 2
    wid = lax.axis_index("s") * nc + lax.axis_index("c")
    n = 32768
    n_chunks = n // 16
    lane = lax.iota(jnp.int32, 16)
    ones16 = jnp.ones((16,), jnp.int32)

    def do_row(r, _):
        row = wid * 4 + r
        pltpu.sync_copy(x_hbm.at[row], xv)

        def level_step(lv, carry):
            prefix_u, kneed = carry
            shift = 24 - 8 * lv
            mask_hi = jnp.where(lv == 0, np.int32(0),
                                lax.shift_left(np.int32(-1),
                                               jnp.minimum(shift + 8, 31)))

            def zero_step(i, _):
                hist[pl.ds(i * 16, 16)] = jnp.zeros((16,), jnp.float32)
                return 0
            lax.fori_loop(0, 256, zero_step, 0)

            def hist_step(i, _):
                s = _keys(xv[pl.ds(i * 16, 16)])
                u = s ^ _IMIN
                match = (u & mask_hi) == prefix_u
                bucket = lax.shift_right_logical(u, shift) & np.int32(0xFF)
                idxv = bucket * 16 + lane
                h = plsc.load_gather(hist, [idxv])
                plsc.store_scatter(hist, [idxv],
                                   h + jnp.where(match, 1.0, 0.0))
                return 0
            lax.fori_loop(0, n_chunks, hist_step, 0, unroll=4)

            def scan_step(i, c):
                cum, bfound, cum_before = c
                b = 255 - i
                cnt = jnp.sum(hist[pl.ds(b * 16, 16)]).astype(jnp.int32)
                cum_new = cum + cnt
                hit = (cum_new >= kneed) & (bfound < 0)
                bfound = jnp.where(hit, b, bfound)
                cum_before = jnp.where(hit, cum, cum_before)
                return cum_new, bfound, cum_before
            _, bsel, cum_before = lax.fori_loop(
                0, 256, scan_step,
                (np.int32(0), np.int32(-1), np.int32(0)))

            prefix_u = prefix_u | lax.shift_left(bsel, shift)
            kneed = kneed - cum_before
            return prefix_u, kneed

        t_u, kneed = lax.fori_loop(0, 4, level_step,
                                   (np.int32(0), np.int32(_K)))
        t_s = t_u ^ _IMIN

        def out_step(i, c_eq):
            v = xv[pl.ds(i * 16, 16)]
            s = _keys(v)
            gt = s > t_s
            eqm = s == t_s
            eqi = jnp.where(eqm, 1, 0)
            pfx = plsc.cumsum(eqi)
            keep = gt | (eqm & ((c_eq + pfx) <= kneed))
            ov[pl.ds(i * 16, 16)] = jnp.where(
                keep, jnp.maximum(v, 0.0), jnp.float32(0.0))
            return c_eq + jnp.sum(eqi)
        lax.fori_loop(0, n_chunks, out_step, np.int32(0), unroll=4)

        pltpu.sync_copy(ov, o_hbm.at[row])
        return 0

    lax.fori_loop(0, 4, do_row, 0)


def kernel(x):
    n_rows, n = x.shape
    mesh = plsc.VectorSubcoreMesh(core_axis_name="c", subcore_axis_name="s",
                                  num_cores=2, num_subcores=16)
    return pl.kernel(
        _sc_body,
        out_type=jax.ShapeDtypeStruct((n_rows, n), jnp.float32),
        mesh=mesh,
        compiler_params=pltpu.CompilerParams(needs_layout_passes=False),
        scratch_types=[
            pltpu.VMEM((n,), jnp.float32),
            pltpu.VMEM((n,), jnp.float32),
            pltpu.VMEM((4096,), jnp.float32),
            pltpu.SemaphoreType.DMA,
        ],
    )(x)
